# baseline (speedup 1.0000x reference)
import functools

import jax
import jax.numpy as jnp
from jax.experimental import pallas as pl
from jax.experimental.pallas import tpu as pltpu

E = 8
TOPK = 2
T = 256
BT = 64
NBPE = T // BT
EBT = NBPE * BT
NP = E * EBT
P2 = TOPK * T


def _routing_kernel(g_ref, be_ref, act_ref, ids_ref, w_ref):
    logits = g_ref[...]
    m = jnp.max(logits, axis=1, keepdims=True)
    p = jnp.exp(logits - m)
    p = p / jnp.sum(p, axis=1, keepdims=True)

    eidx = jax.lax.broadcasted_iota(jnp.int32, (T, E), 1)
    m1 = jnp.max(p, axis=1, keepdims=True)
    a1 = jnp.min(jnp.where(p == m1, eidx, E), axis=1, keepdims=True)
    p2 = jnp.where(eidx == a1, -1.0, p)
    m2 = jnp.max(p2, axis=1, keepdims=True)
    a2 = jnp.min(jnp.where(p2 == m2, eidx, E), axis=1, keepdims=True)
    s = m1 + m2
    w1 = m1 / s
    w2 = m2 / s

    e_pairs = jnp.concatenate([a1, a2], axis=0)
    w_pairs = jnp.concatenate([w1, w2], axis=0)
    tio = jax.lax.broadcasted_iota(jnp.int32, (T, 1), 0).astype(jnp.float32)
    t_pairs = jnp.concatenate([tio, tio], axis=0)

    oh = (e_pairs == jax.lax.broadcasted_iota(jnp.int32, (P2, E), 1))
    ohf = oh.astype(jnp.float32)
    counts = jnp.sum(ohf, axis=0, keepdims=True)

    pr = jax.lax.broadcasted_iota(jnp.int32, (P2, P2), 0)
    pc = jax.lax.broadcasted_iota(jnp.int32, (P2, P2), 1)
    lt = (pc <= pr).astype(jnp.float32)
    incl = jnp.dot(lt, ohf, preferred_element_type=jnp.float32)
    rank = jnp.sum((incl - 1.0) * ohf, axis=1, keepdims=True)
    pos = e_pairs.astype(jnp.float32) * EBT + rank

    slot_iota = jax.lax.broadcasted_iota(jnp.int32, (P2, NP), 1).astype(
        jnp.float32)
    at = (pos == slot_iota).astype(jnp.float32)
    tw = jnp.concatenate([t_pairs, w_pairs], axis=1)
    cdims = (((0,), (0,)), ((), ()))
    idw = jax.lax.dot_general(
        at, tw, cdims, preferred_element_type=jnp.float32)
    ids_ref[...] = idw[:, 0:1]
    w_ref[...] = idw[:, 1:2]

    fio = jax.lax.broadcasted_iota(jnp.int32, (1, E * NBPE), 1)
    f_e = fio // NBPE
    f_j = fio % NBPE
    ohe = (jax.lax.broadcasted_iota(jnp.int32, (E, E * NBPE), 0)
           == f_e).astype(jnp.float32)
    cnt_per = jnp.dot(counts, ohe, preferred_element_type=jnp.float32)
    act_ref[...] = (cnt_per > (f_j * BT).astype(jnp.float32)).astype(jnp.int32)

    used_val = jnp.where(counts > 0,
                         jax.lax.broadcasted_iota(jnp.int32, (1, E), 1), -1)
    er = jax.lax.broadcasted_iota(jnp.int32, (E, E), 0)
    ec = jax.lax.broadcasted_iota(jnp.int32, (E, E), 1)
    masked = jnp.where(ec <= er, jnp.broadcast_to(used_val, (E, E)), -1)
    cm = jnp.max(masked, axis=1, keepdims=True)
    be_ref[...] = jnp.maximum(jnp.transpose(cm), 0)


def _moe_kernel(be_ref, act_ref, ids_ref, w_ref, x_ref, gu_ref, dn_ref,
                out_ref, *, ff):
    e = pl.program_id(0)
    j = pl.program_id(1)

    @pl.when((e == 0) & (j == 0))
    def _init():
        out_ref[...] = jnp.zeros_like(out_ref)

    @pl.when(act_ref[e * NBPE + j] > 0)
    def _compute():
        base = e * EBT + j * BT
        ids = ids_ref[pl.ds(base, BT), :]
        w = w_ref[pl.ds(base, BT), :]
        tcol = jax.lax.broadcasted_iota(jnp.int32, (BT, T), 1).astype(
            jnp.float32)
        perm = (ids == tcol).astype(jnp.float32)
        xg = jnp.dot(perm, x_ref[...],
                     preferred_element_type=jnp.float32)
        wgu = gu_ref[0]
        cdims = (((1,), (1,)), ((), ()))
        gu = jax.lax.dot_general(xg, wgu, cdims,
                                 preferred_element_type=jnp.float32)
        g = gu[:, :ff]
        u = gu[:, ff:]
        h = g * jax.lax.logistic(g) * u
        dn = dn_ref[0]
        y = jax.lax.dot_general(h, dn, cdims,
                                preferred_element_type=jnp.float32)
        y = y * w
        sdims = (((0,), (0,)), ((), ()))
        out_ref[...] += jax.lax.dot_general(
            perm, y, sdims, preferred_element_type=jnp.float32)


def kernel(x, gating_output, gate_up_proj, down_proj):
    t, d = x.shape
    ff2 = gate_up_proj.shape[1]
    ff = ff2 // 2

    be, act, ids, w = pl.pallas_call(
        _routing_kernel,
        out_shape=[
            jax.ShapeDtypeStruct((1, E), jnp.int32),
            jax.ShapeDtypeStruct((1, E * NBPE), jnp.int32),
            jax.ShapeDtypeStruct((NP, 1), jnp.float32),
            jax.ShapeDtypeStruct((NP, 1), jnp.float32),
        ],
    )(gating_output)

    be = be.reshape(E)
    act = act.reshape(E * NBPE)

    grid_spec = pltpu.PrefetchScalarGridSpec(
        num_scalar_prefetch=2,
        grid=(E, NBPE),
        in_specs=[
            pl.BlockSpec((NP, 1), lambda e, j, be_r, act_r: (0, 0)),
            pl.BlockSpec((NP, 1), lambda e, j, be_r, act_r: (0, 0)),
            pl.BlockSpec((t, d), lambda e, j, be_r, act_r: (0, 0)),
            pl.BlockSpec((1, ff2, d), lambda e, j, be_r, act_r: (be_r[e], 0, 0)),
            pl.BlockSpec((1, d, ff), lambda e, j, be_r, act_r: (be_r[e], 0, 0)),
        ],
        out_specs=pl.BlockSpec((t, d), lambda e, j, be_r, act_r: (0, 0)),
    )

    out = pl.pallas_call(
        functools.partial(_moe_kernel, ff=ff),
        grid_spec=grid_spec,
        out_shape=jax.ShapeDtypeStruct((t, d), jnp.float32),
    )(be, act, ids, w, x, gate_up_proj, down_proj)
    return out


# speedup vs baseline: 1.0176x; 1.0176x over previous
import functools

import jax
import jax.numpy as jnp
from jax.experimental import pallas as pl
from jax.experimental.pallas import tpu as pltpu

E = 8
TOPK = 2
T = 256
BT = 64
NBPE = T // BT
EBT = NBPE * BT
NP = E * EBT
P2 = TOPK * T


def _routing_kernel(g_ref, be_ref, act_ref, ids_ref, w_ref):
    logits = g_ref[...]
    m = jnp.max(logits, axis=1, keepdims=True)
    p = jnp.exp(logits - m)
    p = p / jnp.sum(p, axis=1, keepdims=True)

    eidx = jax.lax.broadcasted_iota(jnp.int32, (T, E), 1)
    m1 = jnp.max(p, axis=1, keepdims=True)
    a1 = jnp.min(jnp.where(p == m1, eidx, E), axis=1, keepdims=True)
    p2 = jnp.where(eidx == a1, -1.0, p)
    m2 = jnp.max(p2, axis=1, keepdims=True)
    a2 = jnp.min(jnp.where(p2 == m2, eidx, E), axis=1, keepdims=True)
    s = m1 + m2
    w1 = m1 / s
    w2 = m2 / s

    e_pairs = jnp.concatenate([a1, a2], axis=0)
    w_pairs = jnp.concatenate([w1, w2], axis=0)
    tio = jax.lax.broadcasted_iota(jnp.int32, (T, 1), 0).astype(jnp.float32)
    t_pairs = jnp.concatenate([tio, tio], axis=0)

    oh = (e_pairs == jax.lax.broadcasted_iota(jnp.int32, (P2, E), 1))
    ohf = oh.astype(jnp.float32)
    counts = jnp.sum(ohf, axis=0, keepdims=True)

    pr = jax.lax.broadcasted_iota(jnp.int32, (P2, P2), 0)
    pc = jax.lax.broadcasted_iota(jnp.int32, (P2, P2), 1)
    lt = (pc <= pr).astype(jnp.float32)
    incl = jnp.dot(lt, ohf, preferred_element_type=jnp.float32)
    rank = jnp.sum((incl - 1.0) * ohf, axis=1, keepdims=True)
    pos = e_pairs.astype(jnp.float32) * EBT + rank

    slot_iota = jax.lax.broadcasted_iota(jnp.int32, (P2, NP), 1).astype(
        jnp.float32)
    at = (pos == slot_iota).astype(jnp.float32)
    tw = jnp.concatenate([t_pairs, w_pairs], axis=1)
    cdims = (((0,), (0,)), ((), ()))
    idw = jax.lax.dot_general(
        at, tw, cdims, preferred_element_type=jnp.float32)
    ids_ref[...] = idw[:, 0:1]
    w_ref[...] = idw[:, 1:2]

    fio = jax.lax.broadcasted_iota(jnp.int32, (1, E * NBPE), 1)
    f_e = fio // NBPE
    f_j = fio % NBPE
    ohe = (jax.lax.broadcasted_iota(jnp.int32, (E, E * NBPE), 0)
           == f_e).astype(jnp.float32)
    cnt_per = jnp.dot(counts, ohe, preferred_element_type=jnp.float32)
    act_ref[...] = (cnt_per > (f_j * BT).astype(jnp.float32)).astype(jnp.int32)

    used_val = jnp.where(counts > 0,
                         jax.lax.broadcasted_iota(jnp.int32, (1, E), 1), -1)
    er = jax.lax.broadcasted_iota(jnp.int32, (E, E), 0)
    ec = jax.lax.broadcasted_iota(jnp.int32, (E, E), 1)
    masked = jnp.where(ec <= er, jnp.broadcast_to(used_val, (E, E)), -1)
    cm = jnp.max(masked, axis=1, keepdims=True)
    be_ref[...] = jnp.maximum(jnp.transpose(cm), 0)


def _moe_kernel(be_ref, act_ref, ids_ref, w_ref, x_ref, gu_ref, dn_ref,
                out_ref, *, ff):
    e = pl.program_id(0)
    j = pl.program_id(1)

    @pl.when((e == 0) & (j == 0))
    def _init():
        out_ref[...] = jnp.zeros_like(out_ref)

    @pl.when(act_ref[e * NBPE + j] > 0)
    def _compute():
        base = e * EBT + j * BT
        ids = ids_ref[pl.ds(base, BT), :]
        w = w_ref[pl.ds(base, BT), :]
        tcol = jax.lax.broadcasted_iota(jnp.int32, (BT, T), 1).astype(
            jnp.float32)
        perm = (ids == tcol).astype(jnp.float32)
        xg = jnp.dot(perm, x_ref[...],
                     preferred_element_type=jnp.float32)
        wgu = gu_ref[0]
        cdims = (((1,), (1,)), ((), ()))
        gu = jax.lax.dot_general(xg, wgu, cdims,
                                 preferred_element_type=jnp.float32)
        g = gu[:, :ff]
        u = gu[:, ff:]
        h = g * jax.lax.logistic(g) * u
        dn = dn_ref[0]
        y = jax.lax.dot_general(h, dn, cdims,
                                preferred_element_type=jnp.float32)
        y = y * w
        sdims = (((0,), (0,)), ((), ()))
        out_ref[...] += jax.lax.dot_general(
            perm, y, sdims, preferred_element_type=jnp.float32)


def kernel(x, gating_output, gate_up_proj, down_proj):
    t, d = x.shape
    ff2 = gate_up_proj.shape[1]
    ff = ff2 // 2

    be, act, ids, w = pl.pallas_call(
        _routing_kernel,
        out_shape=[
            jax.ShapeDtypeStruct((1, E), jnp.int32),
            jax.ShapeDtypeStruct((1, E * NBPE), jnp.int32),
            jax.ShapeDtypeStruct((NP, 1), jnp.float32),
            jax.ShapeDtypeStruct((NP, 1), jnp.float32),
        ],
    )(gating_output)

    be = be.reshape(E)
    act = act.reshape(E * NBPE)

    grid_spec = pltpu.PrefetchScalarGridSpec(
        num_scalar_prefetch=2,
        grid=(E, NBPE),
        in_specs=[
            pl.BlockSpec((NP, 1), lambda e, j, be_r, act_r: (0, 0)),
            pl.BlockSpec((NP, 1), lambda e, j, be_r, act_r: (0, 0)),
            pl.BlockSpec((t, d), lambda e, j, be_r, act_r: (0, 0)),
            pl.BlockSpec((1, ff2, d), lambda e, j, be_r, act_r: (e, 0, 0)),
            pl.BlockSpec((1, d, ff), lambda e, j, be_r, act_r: (e, 0, 0)),
        ],
        out_specs=pl.BlockSpec((t, d), lambda e, j, be_r, act_r: (0, 0)),
    )

    out = pl.pallas_call(
        functools.partial(_moe_kernel, ff=ff),
        grid_spec=grid_spec,
        out_shape=jax.ShapeDtypeStruct((t, d), jnp.float32),
    )(be, act, ids, w, x, gate_up_proj, down_proj)
    return out


# speedup vs baseline: 1.3522x; 1.3288x over previous
import functools

import jax
import jax.numpy as jnp
from jax.experimental import pallas as pl
from jax.experimental.pallas import tpu as pltpu

E = 8
TOPK = 2
T = 256
BT = 64
NBPE = T // BT
EBT = NBPE * BT
NP = E * EBT
P2 = TOPK * T


def _routing_kernel(g_ref, act_ref, ids_ref, w_ref):
    logits = g_ref[...]
    m = jnp.max(logits, axis=1, keepdims=True)
    p = jnp.exp(logits - m)
    p = p / jnp.sum(p, axis=1, keepdims=True)

    eidx = jax.lax.broadcasted_iota(jnp.int32, (T, E), 1)
    m1 = jnp.max(p, axis=1, keepdims=True)
    a1 = jnp.min(jnp.where(p == m1, eidx, E), axis=1, keepdims=True)
    p2 = jnp.where(eidx == a1, -1.0, p)
    m2 = jnp.max(p2, axis=1, keepdims=True)
    a2 = jnp.min(jnp.where(p2 == m2, eidx, E), axis=1, keepdims=True)
    s = m1 + m2
    w1 = m1 / s
    w2 = m2 / s

    e_pairs = jnp.concatenate([a1, a2], axis=0)
    w_pairs = jnp.concatenate([w1, w2], axis=0)
    tio = jax.lax.broadcasted_iota(jnp.int32, (T, 1), 0).astype(jnp.float32)
    t_pairs = jnp.concatenate([tio, tio], axis=0)

    oh = (e_pairs == jax.lax.broadcasted_iota(jnp.int32, (P2, E), 1))
    ohf = oh.astype(jnp.float32)
    counts = jnp.sum(ohf, axis=0, keepdims=True)

    pr = jax.lax.broadcasted_iota(jnp.int32, (P2, P2), 0)
    pc = jax.lax.broadcasted_iota(jnp.int32, (P2, P2), 1)
    lt = (pc <= pr).astype(jnp.float32)
    incl = jnp.dot(lt, ohf, preferred_element_type=jnp.float32)
    rank = jnp.sum((incl - 1.0) * ohf, axis=1, keepdims=True)
    pos = e_pairs.astype(jnp.float32) * EBT + rank

    slot_iota = jax.lax.broadcasted_iota(jnp.int32, (P2, NP), 1).astype(
        jnp.float32)
    at = (pos == slot_iota).astype(jnp.float32)
    tw = jnp.concatenate([t_pairs, w_pairs], axis=1)
    cdims = (((0,), (0,)), ((), ()))
    idw = jax.lax.dot_general(
        at, tw, cdims, preferred_element_type=jnp.float32)
    ids_ref[...] = idw[:, 0:1]
    w_ref[...] = idw[:, 1:2]

    fio = jax.lax.broadcasted_iota(jnp.int32, (1, E * NBPE), 1)
    f_e = fio // NBPE
    f_j = fio % NBPE
    ohe = (jax.lax.broadcasted_iota(jnp.int32, (E, E * NBPE), 0)
           == f_e).astype(jnp.float32)
    cnt_per = jnp.dot(counts, ohe, preferred_element_type=jnp.float32)
    act_ref[...] = (cnt_per > (f_j * BT).astype(jnp.float32)).astype(jnp.int32)


def _moe_kernel(act_ref, ids_ref, w_ref, x_ref, gu_hbm, dn_hbm, out_ref,
                gu_buf, dn_buf, gu_sem, dn_sem, *, ff):
    def start_copy(e, slot):
        pltpu.make_async_copy(gu_hbm.at[e], gu_buf.at[slot],
                              gu_sem.at[slot]).start()
        pltpu.make_async_copy(dn_hbm.at[e], dn_buf.at[slot],
                              dn_sem.at[slot]).start()

    def wait_copy(e, slot):
        pltpu.make_async_copy(gu_hbm.at[e], gu_buf.at[slot],
                              gu_sem.at[slot]).wait()
        pltpu.make_async_copy(dn_hbm.at[e], dn_buf.at[slot],
                              dn_sem.at[slot]).wait()

    out_ref[...] = jnp.zeros_like(out_ref)
    x = x_ref[...]

    @pl.when(act_ref[0] > 0)
    def _():
        start_copy(0, 0)

    for e in range(E):
        used = act_ref[e * NBPE] > 0
        if e + 1 < E:
            nxt = act_ref[(e + 1) * NBPE] > 0

            @pl.when(nxt)
            def _(e=e):
                start_copy(e + 1, (e + 1) % 2)

        @pl.when(used)
        def _(e=e):
            wait_copy(e, e % 2)

        for j in range(NBPE):
            @pl.when(act_ref[e * NBPE + j] > 0)
            def _(e=e, j=j):
                slot = e % 2
                base = e * EBT + j * BT
                ids = ids_ref[base:base + BT, :]
                w = w_ref[base:base + BT, :]
                tcol = jax.lax.broadcasted_iota(
                    jnp.int32, (BT, T), 1).astype(jnp.float32)
                perm = (ids == tcol).astype(jnp.float32)
                xg = jnp.dot(perm, x,
                             preferred_element_type=jnp.float32)
                wgu = gu_buf[slot]
                cdims = (((1,), (1,)), ((), ()))
                gu = jax.lax.dot_general(
                    xg, wgu, cdims,
                    preferred_element_type=jnp.float32)
                g = gu[:, :ff]
                u = gu[:, ff:]
                h = g * jax.lax.logistic(g) * u
                dn = dn_buf[slot]
                y = jax.lax.dot_general(
                    h, dn, cdims,
                    preferred_element_type=jnp.float32)
                y = y * w
                sdims = (((0,), (0,)), ((), ()))
                out_ref[...] += jax.lax.dot_general(
                    perm, y, sdims,
                    preferred_element_type=jnp.float32)


def kernel(x, gating_output, gate_up_proj, down_proj):
    t, d = x.shape
    ff2 = gate_up_proj.shape[1]
    ff = ff2 // 2

    act, ids, w = pl.pallas_call(
        _routing_kernel,
        out_shape=[
            jax.ShapeDtypeStruct((1, E * NBPE), jnp.int32),
            jax.ShapeDtypeStruct((NP, 1), jnp.float32),
            jax.ShapeDtypeStruct((NP, 1), jnp.float32),
        ],
    )(gating_output)

    act = act.reshape(E * NBPE)

    grid_spec = pltpu.PrefetchScalarGridSpec(
        num_scalar_prefetch=1,
        grid=(1,),
        in_specs=[
            pl.BlockSpec((NP, 1), lambda i, act_r: (0, 0)),
            pl.BlockSpec((NP, 1), lambda i, act_r: (0, 0)),
            pl.BlockSpec((t, d), lambda i, act_r: (0, 0)),
            pl.BlockSpec(memory_space=pltpu.MemorySpace.HBM),
            pl.BlockSpec(memory_space=pltpu.MemorySpace.HBM),
        ],
        out_specs=pl.BlockSpec((t, d), lambda i, act_r: (0, 0)),
        scratch_shapes=[
            pltpu.VMEM((2, ff2, d), jnp.float32),
            pltpu.VMEM((2, d, ff), jnp.float32),
            pltpu.SemaphoreType.DMA((2,)),
            pltpu.SemaphoreType.DMA((2,)),
        ],
    )

    out = pl.pallas_call(
        functools.partial(_moe_kernel, ff=ff),
        grid_spec=grid_spec,
        out_shape=jax.ShapeDtypeStruct((t, d), jnp.float32),
    )(act, ids, w, x, gate_up_proj, down_proj)
    return out


# speedup vs baseline: 1.3696x; 1.0128x over previous
import functools

import jax
import jax.numpy as jnp
from jax.experimental import pallas as pl
from jax.experimental.pallas import tpu as pltpu

E = 8
TOPK = 2
T = 256
BT = 64
NBPE = T // BT
EBT = NBPE * BT
NP = E * EBT
P2 = TOPK * T


def _routing_kernel(g_ref, act_ref, ids_ref, w_ref):
    logits = g_ref[...]
    m = jnp.max(logits, axis=1, keepdims=True)
    p = jnp.exp(logits - m)
    p = p / jnp.sum(p, axis=1, keepdims=True)

    eidx = jax.lax.broadcasted_iota(jnp.int32, (T, E), 1)
    m1 = jnp.max(p, axis=1, keepdims=True)
    a1 = jnp.min(jnp.where(p == m1, eidx, E), axis=1, keepdims=True)
    p2 = jnp.where(eidx == a1, -1.0, p)
    m2 = jnp.max(p2, axis=1, keepdims=True)
    a2 = jnp.min(jnp.where(p2 == m2, eidx, E), axis=1, keepdims=True)
    s = m1 + m2
    w1 = m1 / s
    w2 = m2 / s

    e_pairs = jnp.concatenate([a1, a2], axis=0)
    w_pairs = jnp.concatenate([w1, w2], axis=0)
    tio = jax.lax.broadcasted_iota(jnp.int32, (T, 1), 0).astype(jnp.float32)
    t_pairs = jnp.concatenate([tio, tio], axis=0)

    oh = (e_pairs == jax.lax.broadcasted_iota(jnp.int32, (P2, E), 1))
    ohf = oh.astype(jnp.float32)
    counts = jnp.sum(ohf, axis=0, keepdims=True)

    pr = jax.lax.broadcasted_iota(jnp.int32, (P2, P2), 0)
    pc = jax.lax.broadcasted_iota(jnp.int32, (P2, P2), 1)
    lt = (pc <= pr).astype(jnp.float32)
    incl = jnp.dot(lt, ohf, preferred_element_type=jnp.float32)
    rank = jnp.sum((incl - 1.0) * ohf, axis=1, keepdims=True)
    pos = e_pairs.astype(jnp.float32) * EBT + rank

    slot_iota = jax.lax.broadcasted_iota(jnp.int32, (P2, NP), 1).astype(
        jnp.float32)
    at = (pos == slot_iota).astype(jnp.float32)
    tw = jnp.concatenate([t_pairs, w_pairs], axis=1)
    cdims = (((0,), (0,)), ((), ()))
    idw = jax.lax.dot_general(
        at, tw, cdims, preferred_element_type=jnp.float32)
    ids_ref[...] = idw[:, 0:1]
    w_ref[...] = idw[:, 1:2]

    fio = jax.lax.broadcasted_iota(jnp.int32, (1, E * NBPE), 1)
    f_e = fio // NBPE
    f_j = fio % NBPE
    ohe = (jax.lax.broadcasted_iota(jnp.int32, (E, E * NBPE), 0)
           == f_e).astype(jnp.float32)
    cnt_per = jnp.dot(counts, ohe, preferred_element_type=jnp.float32)
    act_ref[...] = (cnt_per > (f_j * BT).astype(jnp.float32)).astype(jnp.int32)


def _moe_kernel(act_ref, ids_ref, w_ref, x_ref, gu_hbm, dn_hbm, out_ref,
                gu_buf, dn_buf, gu_sem, dn_sem, *, ff):
    def start_copy(e, slot):
        pltpu.make_async_copy(gu_hbm.at[e], gu_buf.at[slot],
                              gu_sem.at[slot]).start()
        pltpu.make_async_copy(dn_hbm.at[e], dn_buf.at[slot],
                              dn_sem.at[slot]).start()

    def wait_copy(e, slot):
        pltpu.make_async_copy(gu_hbm.at[e], gu_buf.at[slot],
                              gu_sem.at[slot]).wait()
        pltpu.make_async_copy(dn_hbm.at[e], dn_buf.at[slot],
                              dn_sem.at[slot]).wait()

    out_ref[...] = jnp.zeros_like(out_ref)
    x = x_ref[...]

    @pl.when(act_ref[0] > 0)
    def _():
        start_copy(0, 0)

    for e in range(E):
        used = act_ref[e * NBPE] > 0
        if e + 1 < E:
            nxt = act_ref[(e + 1) * NBPE] > 0

            @pl.when(nxt)
            def _(e=e):
                start_copy(e + 1, (e + 1) % 2)

        @pl.when(used)
        def _(e=e):
            wait_copy(e, e % 2)

        for j in range(NBPE):
            @pl.when(act_ref[e * NBPE + j] > 0)
            def _(e=e, j=j):
                slot = e % 2
                base = e * EBT + j * BT
                ids = ids_ref[base:base + BT, :]
                w = w_ref[base:base + BT, :]
                tcol = jax.lax.broadcasted_iota(
                    jnp.int32, (BT, T), 1).astype(jnp.float32)
                perm = (ids == tcol).astype(jnp.float32)
                xg = jnp.dot(perm, x,
                             preferred_element_type=jnp.float32)
                wgu = gu_buf[slot]
                cdims = (((1,), (1,)), ((), ()))
                gu = jax.lax.dot_general(
                    xg.astype(jnp.bfloat16), wgu, cdims,
                    preferred_element_type=jnp.float32)
                g = gu[:, :ff]
                u = gu[:, ff:]
                h = g * jax.lax.logistic(g) * u
                dn = dn_buf[slot]
                y = jax.lax.dot_general(
                    h.astype(jnp.bfloat16), dn, cdims,
                    preferred_element_type=jnp.float32)
                y = y * w
                sdims = (((0,), (0,)), ((), ()))
                out_ref[...] += jax.lax.dot_general(
                    perm, y, sdims,
                    preferred_element_type=jnp.float32)


def kernel(x, gating_output, gate_up_proj, down_proj):
    t, d = x.shape
    ff2 = gate_up_proj.shape[1]
    ff = ff2 // 2

    act, ids, w = pl.pallas_call(
        _routing_kernel,
        out_shape=[
            jax.ShapeDtypeStruct((1, E * NBPE), jnp.int32),
            jax.ShapeDtypeStruct((NP, 1), jnp.float32),
            jax.ShapeDtypeStruct((NP, 1), jnp.float32),
        ],
    )(gating_output)

    act = act.reshape(E * NBPE)

    grid_spec = pltpu.PrefetchScalarGridSpec(
        num_scalar_prefetch=1,
        grid=(1,),
        in_specs=[
            pl.BlockSpec((NP, 1), lambda i, act_r: (0, 0)),
            pl.BlockSpec((NP, 1), lambda i, act_r: (0, 0)),
            pl.BlockSpec((t, d), lambda i, act_r: (0, 0)),
            pl.BlockSpec(memory_space=pltpu.MemorySpace.HBM),
            pl.BlockSpec(memory_space=pltpu.MemorySpace.HBM),
        ],
        out_specs=pl.BlockSpec((t, d), lambda i, act_r: (0, 0)),
        scratch_shapes=[
            pltpu.VMEM((2, ff2, d), jnp.float32),
            pltpu.VMEM((2, d, ff), jnp.float32),
            pltpu.SemaphoreType.DMA((2,)),
            pltpu.SemaphoreType.DMA((2,)),
        ],
    )

    out = pl.pallas_call(
        functools.partial(_moe_kernel, ff=ff),
        grid_spec=grid_spec,
        out_shape=jax.ShapeDtypeStruct((t, d), jnp.float32),
    )(act, ids, w, x, gate_up_proj, down_proj)
    return out


# speedup vs baseline: 1.7096x; 1.2483x over previous
import functools

import jax
import jax.numpy as jnp
from jax.experimental import pallas as pl
from jax.experimental.pallas import tpu as pltpu

E = 8
TOPK = 2
T = 256
BT = 64
NBPE = T // BT
EBT = NBPE * BT
NP = E * EBT
P2 = TOPK * T


def _routing_kernel(g_ref, act_ref, ids_ref, w_ref):
    logits = g_ref[...]
    m = jnp.max(logits, axis=1, keepdims=True)
    p = jnp.exp(logits - m)
    p = p / jnp.sum(p, axis=1, keepdims=True)

    eidx = jax.lax.broadcasted_iota(jnp.int32, (T, E), 1)
    m1 = jnp.max(p, axis=1, keepdims=True)
    a1 = jnp.min(jnp.where(p == m1, eidx, E), axis=1, keepdims=True)
    p2 = jnp.where(eidx == a1, -1.0, p)
    m2 = jnp.max(p2, axis=1, keepdims=True)
    a2 = jnp.min(jnp.where(p2 == m2, eidx, E), axis=1, keepdims=True)
    s = m1 + m2
    w1 = m1 / s
    w2 = m2 / s

    e_pairs = jnp.concatenate([a1, a2], axis=0)
    w_pairs = jnp.concatenate([w1, w2], axis=0)
    tio = jax.lax.broadcasted_iota(jnp.int32, (T, 1), 0).astype(jnp.float32)
    t_pairs = jnp.concatenate([tio, tio], axis=0)

    oh = (e_pairs == jax.lax.broadcasted_iota(jnp.int32, (P2, E), 1))
    ohf = oh.astype(jnp.float32)
    counts = jnp.sum(ohf, axis=0, keepdims=True)

    pr = jax.lax.broadcasted_iota(jnp.int32, (P2, P2), 0)
    pc = jax.lax.broadcasted_iota(jnp.int32, (P2, P2), 1)
    lt = (pc <= pr).astype(jnp.float32)
    incl = jnp.dot(lt, ohf, preferred_element_type=jnp.float32)
    rank = jnp.sum((incl - 1.0) * ohf, axis=1, keepdims=True)
    pos = e_pairs.astype(jnp.float32) * EBT + rank

    slot_iota = jax.lax.broadcasted_iota(jnp.int32, (P2, NP), 1).astype(
        jnp.float32)
    at = (pos == slot_iota).astype(jnp.float32)
    tw = jnp.concatenate([t_pairs, w_pairs], axis=1)
    cdims = (((0,), (0,)), ((), ()))
    idw = jax.lax.dot_general(
        at, tw, cdims, preferred_element_type=jnp.float32)
    ids_ref[...] = idw[:, 0:1]
    w_ref[...] = idw[:, 1:2]

    fio = jax.lax.broadcasted_iota(jnp.int32, (1, E * NBPE), 1)
    f_e = fio // NBPE
    f_j = fio % NBPE
    ohe = (jax.lax.broadcasted_iota(jnp.int32, (E, E * NBPE), 0)
           == f_e).astype(jnp.float32)
    cnt_per = jnp.dot(counts, ohe, preferred_element_type=jnp.float32)
    act_ref[...] = (cnt_per > (f_j * BT).astype(jnp.float32)).astype(jnp.int32)


def _moe_kernel(act_ref, ids_ref, w_ref, x_ref, gu_hbm, dn_hbm, out_ref,
                gu_buf, dn_buf, gu_sem, dn_sem, *, ff):
    def start_copy(e, slot):
        pltpu.make_async_copy(gu_hbm.at[e], gu_buf.at[slot],
                              gu_sem.at[slot]).start()
        pltpu.make_async_copy(dn_hbm.at[e], dn_buf.at[slot],
                              dn_sem.at[slot]).start()

    def wait_copy(e, slot):
        pltpu.make_async_copy(gu_hbm.at[e], gu_buf.at[slot],
                              gu_sem.at[slot]).wait()
        pltpu.make_async_copy(dn_hbm.at[e], dn_buf.at[slot],
                              dn_sem.at[slot]).wait()

    out_ref[...] = jnp.zeros_like(out_ref)
    x = x_ref[...]

    @pl.when(act_ref[0] > 0)
    def _():
        start_copy(0, 0)

    for e in range(E):
        used = act_ref[e * NBPE] > 0
        if e + 1 < E:
            nxt = act_ref[(e + 1) * NBPE] > 0

            @pl.when(nxt)
            def _(e=e):
                start_copy(e + 1, (e + 1) % 2)

        @pl.when(used)
        def _(e=e):
            wait_copy(e, e % 2)
            slot = e % 2
            base = e * EBT
            ids = ids_ref[base:base + EBT, :]
            w = w_ref[base:base + EBT, :]
            tcol = jax.lax.broadcasted_iota(
                jnp.int32, (EBT, T), 1).astype(jnp.float32)
            perm = (ids == tcol).astype(jnp.float32)
            xg = jnp.dot(perm, x,
                         preferred_element_type=jnp.float32)
            wgu = gu_buf[slot]
            cdims = (((1,), (1,)), ((), ()))
            gu = jax.lax.dot_general(
                xg.astype(jnp.bfloat16), wgu, cdims,
                preferred_element_type=jnp.float32)
            g = gu[:, :ff]
            u = gu[:, ff:]
            h = g * jax.lax.logistic(g) * u
            dn = dn_buf[slot]
            y = jax.lax.dot_general(
                h.astype(jnp.bfloat16), dn, cdims,
                preferred_element_type=jnp.float32)
            y = y * w
            sdims = (((0,), (0,)), ((), ()))
            out_ref[...] += jax.lax.dot_general(
                perm, y, sdims,
                preferred_element_type=jnp.float32)


def kernel(x, gating_output, gate_up_proj, down_proj):
    t, d = x.shape
    ff2 = gate_up_proj.shape[1]
    ff = ff2 // 2

    act, ids, w = pl.pallas_call(
        _routing_kernel,
        out_shape=[
            jax.ShapeDtypeStruct((1, E * NBPE), jnp.int32),
            jax.ShapeDtypeStruct((NP, 1), jnp.float32),
            jax.ShapeDtypeStruct((NP, 1), jnp.float32),
        ],
    )(gating_output)

    act = act.reshape(E * NBPE)

    grid_spec = pltpu.PrefetchScalarGridSpec(
        num_scalar_prefetch=1,
        grid=(1,),
        in_specs=[
            pl.BlockSpec((NP, 1), lambda i, act_r: (0, 0)),
            pl.BlockSpec((NP, 1), lambda i, act_r: (0, 0)),
            pl.BlockSpec((t, d), lambda i, act_r: (0, 0)),
            pl.BlockSpec(memory_space=pltpu.MemorySpace.HBM),
            pl.BlockSpec(memory_space=pltpu.MemorySpace.HBM),
        ],
        out_specs=pl.BlockSpec((t, d), lambda i, act_r: (0, 0)),
        scratch_shapes=[
            pltpu.VMEM((2, ff2, d), jnp.float32),
            pltpu.VMEM((2, d, ff), jnp.float32),
            pltpu.SemaphoreType.DMA((2,)),
            pltpu.SemaphoreType.DMA((2,)),
        ],
    )

    out = pl.pallas_call(
        functools.partial(_moe_kernel, ff=ff),
        grid_spec=grid_spec,
        out_shape=jax.ShapeDtypeStruct((t, d), jnp.float32),
    )(act, ids, w, x, gate_up_proj, down_proj)
    return out


# speedup vs baseline: 2.0035x; 1.1719x over previous
import functools

import jax
import jax.numpy as jnp
from jax.experimental import pallas as pl
from jax.experimental.pallas import tpu as pltpu

E = 8
TOPK = 2
T = 256
EBT = T
NP = E * EBT
P2 = TOPK * T


def _routing(logits):
    m = jnp.max(logits, axis=1, keepdims=True)
    p = jnp.exp(logits - m)
    p = p / jnp.sum(p, axis=1, keepdims=True)

    eidx = jax.lax.broadcasted_iota(jnp.int32, (T, E), 1)
    m1 = jnp.max(p, axis=1, keepdims=True)
    a1 = jnp.min(jnp.where(p == m1, eidx, E), axis=1, keepdims=True)
    p2 = jnp.where(eidx == a1, -1.0, p)
    m2 = jnp.max(p2, axis=1, keepdims=True)
    a2 = jnp.min(jnp.where(p2 == m2, eidx, E), axis=1, keepdims=True)
    s = m1 + m2
    w1 = m1 / s
    w2 = m2 / s

    e_pairs = jnp.concatenate([a1, a2], axis=0)
    w_pairs = jnp.concatenate([w1, w2], axis=0)
    tio = jax.lax.broadcasted_iota(jnp.int32, (T, 1), 0).astype(jnp.float32)
    t_pairs = jnp.concatenate([tio, tio], axis=0)

    oh = (e_pairs == jax.lax.broadcasted_iota(jnp.int32, (P2, E), 1))
    ohf = oh.astype(jnp.float32)

    pr = jax.lax.broadcasted_iota(jnp.int32, (P2, P2), 0)
    pc = jax.lax.broadcasted_iota(jnp.int32, (P2, P2), 1)
    lt = (pc <= pr).astype(jnp.float32)
    incl = jnp.dot(lt, ohf, preferred_element_type=jnp.float32)
    rank = jnp.sum((incl - 1.0) * ohf, axis=1, keepdims=True)
    pos = e_pairs.astype(jnp.float32) * EBT + rank

    slot_iota = jax.lax.broadcasted_iota(jnp.int32, (P2, NP), 1).astype(
        jnp.float32)
    at = (pos == slot_iota).astype(jnp.float32)
    tw = jnp.concatenate([t_pairs, w_pairs], axis=1)
    cdims = (((0,), (0,)), ((), ()))
    idw = jax.lax.dot_general(
        at, tw, cdims, preferred_element_type=jnp.float32)
    return idw[:, 0:1], idw[:, 1:2]


def _moe_kernel(g_ref, x_ref, gu_hbm, dn_hbm, out_ref,
                gu_buf, dn_buf, gu_sem, dn_sem, *, ff):
    def gu_copy(e, slot):
        return pltpu.make_async_copy(gu_hbm.at[e], gu_buf.at[slot],
                                     gu_sem.at[slot])

    def dn_copy(e, slot):
        return pltpu.make_async_copy(dn_hbm.at[e], dn_buf.at[slot],
                                     dn_sem.at[slot])

    gu_copy(0, 0).start()
    dn_copy(0, 0).start()
    gu_copy(1, 1).start()
    dn_copy(1, 1).start()

    ids_all, w_all = _routing(g_ref[...])
    x = x_ref[...]
    tcol = jax.lax.broadcasted_iota(jnp.int32, (EBT, T), 1).astype(
        jnp.float32)
    cdims = (((1,), (1,)), ((), ()))
    sdims = (((0,), (0,)), ((), ()))

    for e in range(E):
        slot = e % 2
        base = e * EBT
        ids = ids_all[base:base + EBT, :]
        w = w_all[base:base + EBT, :]
        perm = (ids == tcol).astype(jnp.float32)
        xg = jnp.dot(perm, x,
                     preferred_element_type=jnp.float32)
        gu_copy(e, slot).wait()
        wgu = gu_buf[slot]
        gu = jax.lax.dot_general(
            xg.astype(jnp.bfloat16), wgu, cdims,
            preferred_element_type=jnp.float32)
        g = gu[:, :ff]
        u = gu[:, ff:]
        h = g * jax.lax.logistic(g) * u
        dn_copy(e, slot).wait()
        dn = dn_buf[slot]
        y = jax.lax.dot_general(
            h.astype(jnp.bfloat16), dn, cdims,
            preferred_element_type=jnp.float32)
        y = y * w
        if e + 2 < E:
            gu_copy(e + 2, slot).start()
            dn_copy(e + 2, slot).start()
        contrib = jax.lax.dot_general(
            perm, y, sdims, preferred_element_type=jnp.float32)
        if e == 0:
            out_ref[...] = contrib
        else:
            out_ref[...] += contrib


def kernel(x, gating_output, gate_up_proj, down_proj):
    t, d = x.shape
    ff2 = gate_up_proj.shape[1]
    ff = ff2 // 2

    out = pl.pallas_call(
        functools.partial(_moe_kernel, ff=ff),
        in_specs=[
            pl.BlockSpec(memory_space=pltpu.MemorySpace.VMEM),
            pl.BlockSpec(memory_space=pltpu.MemorySpace.VMEM),
            pl.BlockSpec(memory_space=pltpu.MemorySpace.HBM),
            pl.BlockSpec(memory_space=pltpu.MemorySpace.HBM),
        ],
        out_specs=pl.BlockSpec(memory_space=pltpu.MemorySpace.VMEM),
        scratch_shapes=[
            pltpu.VMEM((2, ff2, d), jnp.float32),
            pltpu.VMEM((2, d, ff), jnp.float32),
            pltpu.SemaphoreType.DMA((2,)),
            pltpu.SemaphoreType.DMA((2,)),
        ],
        out_shape=jax.ShapeDtypeStruct((t, d), jnp.float32),
    )(gating_output, x, gate_up_proj, down_proj)
    return out
